# Initial kernel scaffold; baseline (speedup 1.0000x reference)
import jax
import jax.numpy as jnp
from jax.experimental import pallas as pl


def kernel(x, route_W, route_b, W1, b1, W2, b2):
    raise NotImplementedError("write your pallas kernel here")


# speedup vs baseline: 3.0059x; 3.0059x over previous
import functools

import jax
import jax.numpy as jnp
from jax import lax
from jax.experimental import pallas as pl
from jax.experimental.pallas import tpu as pltpu
from jax.experimental.pallas import tpu_sc as plsc

_S, _DIM, _E, _FF = 2048, 1024, 8, 2048
_BLK = 128
_NB = _S // _BLK + _E
_EP = 128
_NC, _NS = 2, 16
_NW = _NC * _NS
_RPW = _S // _NW


def _router_body(x_ref, w_ref, b_ref, pos_ref, be_ref, nact_ref):
    logits = jnp.dot(x_ref[...].astype(jnp.bfloat16),
                     w_ref[...].astype(jnp.bfloat16),
                     preferred_element_type=jnp.float32) + b_ref[...]
    lane = lax.broadcasted_iota(jnp.int32, (_S, _EP), 1)
    logits = jnp.where(lane < _E, logits, -1e30)
    mx = jnp.max(logits, axis=1, keepdims=True)
    eid = jnp.min(jnp.where(logits == mx, lane, _EP), axis=1, keepdims=True)
    onehot = jnp.where(lane == eid, 1.0, 0.0)

    tri = jnp.where(
        lax.broadcasted_iota(jnp.int32, (_BLK, _BLK), 0)
        >= lax.broadcasted_iota(jnp.int32, (_BLK, _BLK), 1), 1.0, 0.0)

    carry = jnp.zeros((1, _EP), jnp.float32)
    chunks = []
    for g in range(_S // _BLK):
        blk = lax.slice(onehot, (g * _BLK, 0), ((g + 1) * _BLK, _EP))
        c = jnp.dot(tri, blk, preferred_element_type=jnp.float32) + carry
        chunks.append(c)
        carry = lax.slice(c, (_BLK - 1, 0), (_BLK, _EP))
    csum = jnp.concatenate(chunks, axis=0)

    rank = jnp.sum(onehot * csum, axis=1, keepdims=True).astype(jnp.int32) - 1
    counts = lax.slice(csum, (_S - 1, 0), (_S, _EP)).astype(jnp.int32)
    lane_r = lax.broadcasted_iota(jnp.int32, (1, _EP), 1)
    padded = jnp.where(lane_r < _E, ((counts + _BLK - 1) // _BLK) * _BLK, 0)
    triu = jnp.where(
        lax.broadcasted_iota(jnp.int32, (_EP, _EP), 0)
        <= lax.broadcasted_iota(jnp.int32, (_EP, _EP), 1), 1.0, 0.0)
    cum_end = jnp.dot(padded.astype(jnp.float32), triu,
                      preferred_element_type=jnp.float32).astype(jnp.int32)
    off = (cum_end - padded).astype(jnp.float32)
    pos_ref[...] = (jnp.sum(onehot * off, axis=1, keepdims=True).astype(jnp.int32)
                    + rank)

    rowi = lax.broadcasted_iota(jnp.int32, (_NB, _EP), 0) * _BLK
    lane_b = lax.broadcasted_iota(jnp.int32, (_NB, _EP), 1)
    be = jnp.sum(jnp.where((lane_b < _E) & (cum_end <= rowi), 1, 0),
                 axis=1, keepdims=True)
    be_ref[...] = jnp.minimum(be, _E - 1).astype(jnp.int32)
    nact_ref[0, 0] = jnp.sum(jnp.where(lane_r < _E, padded, 0)) // _BLK


def _make_router(interpret=False):
    return pl.pallas_call(
        _router_body,
        out_shape=(
            jax.ShapeDtypeStruct((_S, 1), jnp.int32),
            jax.ShapeDtypeStruct((_NB, 1), jnp.int32),
            jax.ShapeDtypeStruct((1, 1), jnp.int32),
        ),
        out_specs=(
            pl.BlockSpec(memory_space=pltpu.VMEM),
            pl.BlockSpec(memory_space=pltpu.VMEM),
            pl.BlockSpec(memory_space=pltpu.SMEM),
        ),
        interpret=interpret,
    )


def _ffn_body(be_ref, nact_ref, x_ref, w1_ref, b1_ref, w2_ref, b2_ref, o_ref):
    i = pl.program_id(0)

    @pl.when(i < nact_ref[0])
    def _():
        xb = x_ref[...].astype(jnp.bfloat16)
        h = jnp.dot(xb, w1_ref[0], preferred_element_type=jnp.float32)
        h = h + b1_ref[0]
        h = 0.5 * h * (1.0 + lax.erf(h * 0.7071067811865476))
        o = jnp.dot(h.astype(jnp.bfloat16), w2_ref[0],
                    preferred_element_type=jnp.float32)
        o_ref[...] = o + b2_ref[0]


def _make_ffn(interpret=False):
    grid_spec = pltpu.PrefetchScalarGridSpec(
        num_scalar_prefetch=2,
        grid=(_NB,),
        in_specs=[
            pl.BlockSpec((_BLK, _DIM), lambda i, be, na: (i, 0)),
            pl.BlockSpec((1, _DIM, _FF), lambda i, be, na: (be[i], 0, 0)),
            pl.BlockSpec((1, 1, _FF), lambda i, be, na: (be[i], 0, 0)),
            pl.BlockSpec((1, _FF, _DIM), lambda i, be, na: (be[i], 0, 0)),
            pl.BlockSpec((1, 1, _DIM), lambda i, be, na: (be[i], 0, 0)),
        ],
        out_specs=pl.BlockSpec((_BLK, _DIM), lambda i, be, na: (i, 0)),
    )
    return pl.pallas_call(
        _ffn_body,
        grid_spec=grid_spec,
        out_shape=jax.ShapeDtypeStruct((_NB * _BLK, _DIM), jnp.float32),
        interpret=interpret,
    )


@functools.cache
def _make_sc_kernels():
    mesh = plsc.VectorSubcoreMesh(core_axis_name="c", subcore_axis_name="s",
                                  num_cores=_NC)
    scratch = [
        pltpu.VMEM((_RPW,), jnp.int32),
        pltpu.VMEM((_RPW, _DIM), jnp.float32),
        pltpu.SemaphoreType.DMA,
    ]

    @functools.partial(
        pl.kernel, mesh=mesh,
        out_type=jax.ShapeDtypeStruct((_NB * _BLK, _DIM), jnp.float32),
        scratch_types=scratch,
    )
    def _sc_scatter_rows(x_hbm, pos_hbm, xs_hbm, idx_v, rows_v, sem):
        wid = lax.axis_index("s") * _NC + lax.axis_index("c")
        base = wid * _RPW
        pltpu.sync_copy(pos_hbm.at[pl.ds(base, _RPW)], idx_v)
        pltpu.sync_copy(x_hbm.at[pl.ds(base, _RPW)], rows_v)
        pltpu.async_copy(rows_v, xs_hbm.at[idx_v], sem).wait()

    @functools.partial(
        pl.kernel, mesh=mesh,
        out_type=jax.ShapeDtypeStruct((_S, _DIM), jnp.float32),
        scratch_types=scratch,
    )
    def _sc_gather_rows(os_hbm, pos_hbm, out_hbm, idx_v, rows_v, sem):
        wid = lax.axis_index("s") * _NC + lax.axis_index("c")
        base = wid * _RPW
        pltpu.sync_copy(pos_hbm.at[pl.ds(base, _RPW)], idx_v)
        pltpu.async_copy(os_hbm.at[idx_v], rows_v, sem).wait()
        pltpu.sync_copy(rows_v, out_hbm.at[pl.ds(base, _RPW)])

    return _sc_scatter_rows, _sc_gather_rows


def kernel(x, route_W, route_b, W1, b1, W2, b2):
    x2 = x.reshape(_S, _DIM)
    wr = jnp.pad(route_W, ((0, 0), (0, _EP - _E)))
    rb = jnp.pad(route_b, (0, _EP - _E))[None, :]
    pos2, be2, nact2 = _make_router()(x2, wr, rb)
    pos = pos2.reshape(_S)
    be = be2.reshape(_NB)
    nact = nact2.reshape(1)
    sc_scatter, sc_gather = _make_sc_kernels()
    xs = sc_scatter(x2, pos)
    os_ = _make_ffn()(be, nact, xs,
                      W1.astype(jnp.bfloat16), b1.reshape(_E, 1, _FF),
                      W2.astype(jnp.bfloat16), b2.reshape(_E, 1, _DIM))
    out = sc_gather(os_, pos)
    return out.reshape(x.shape)


# speedup vs baseline: 4.0937x; 1.3619x over previous
import functools

import jax
import jax.numpy as jnp
from jax import lax
from jax.experimental import pallas as pl
from jax.experimental.pallas import tpu as pltpu
from jax.experimental.pallas import tpu_sc as plsc

_S, _DIM, _E, _FF = 2048, 1024, 8, 2048
_BLK = 128
_NB = _S // _BLK + _E
_EP = 128
_NC, _NS = 2, 16
_NW = _NC * _NS
_RPW = _S // _NW


def _router_body(x_ref, w_ref, b_ref, pos_ref, be_ref, nact_ref):
    logits = jnp.dot(x_ref[...].astype(jnp.bfloat16),
                     w_ref[...].astype(jnp.bfloat16),
                     preferred_element_type=jnp.float32) + b_ref[...]
    lane = lax.broadcasted_iota(jnp.int32, (_S, _EP), 1)
    logits = jnp.where(lane < _E, logits, -1e30)
    mx = jnp.max(logits, axis=1, keepdims=True)
    eid = jnp.min(jnp.where(logits == mx, lane, _EP), axis=1, keepdims=True)
    onehot = jnp.where(lane == eid, 1.0, 0.0)

    tri = jnp.where(
        lax.broadcasted_iota(jnp.int32, (_BLK, _BLK), 0)
        >= lax.broadcasted_iota(jnp.int32, (_BLK, _BLK), 1), 1.0, 0.0)

    carry = jnp.zeros((1, _EP), jnp.float32)
    chunks = []
    for g in range(_S // _BLK):
        blk = lax.slice(onehot, (g * _BLK, 0), ((g + 1) * _BLK, _EP))
        c = jnp.dot(tri, blk, preferred_element_type=jnp.float32) + carry
        chunks.append(c)
        carry = lax.slice(c, (_BLK - 1, 0), (_BLK, _EP))
    csum = jnp.concatenate(chunks, axis=0)

    rank = jnp.sum(onehot * csum, axis=1, keepdims=True).astype(jnp.int32) - 1
    counts = lax.slice(csum, (_S - 1, 0), (_S, _EP)).astype(jnp.int32)
    lane_r = lax.broadcasted_iota(jnp.int32, (1, _EP), 1)
    padded = jnp.where(lane_r < _E, ((counts + _BLK - 1) // _BLK) * _BLK, 0)
    triu = jnp.where(
        lax.broadcasted_iota(jnp.int32, (_EP, _EP), 0)
        <= lax.broadcasted_iota(jnp.int32, (_EP, _EP), 1), 1.0, 0.0)
    cum_end = jnp.dot(padded.astype(jnp.float32), triu,
                      preferred_element_type=jnp.float32).astype(jnp.int32)
    off = (cum_end - padded).astype(jnp.float32)
    pos_ref[...] = (jnp.sum(onehot * off, axis=1, keepdims=True).astype(jnp.int32)
                    + rank)

    rowi = lax.broadcasted_iota(jnp.int32, (_NB, _EP), 0) * _BLK
    lane_b = lax.broadcasted_iota(jnp.int32, (_NB, _EP), 1)
    be = jnp.sum(jnp.where((lane_b < _E) & (cum_end <= rowi), 1, 0),
                 axis=1, keepdims=True)
    be_ref[...] = jnp.minimum(be, _E - 1).astype(jnp.int32)
    nact_ref[0, 0] = jnp.sum(jnp.where(lane_r < _E, padded, 0)) // _BLK


def _make_router(interpret=False):
    return pl.pallas_call(
        _router_body,
        out_shape=(
            jax.ShapeDtypeStruct((_S, 1), jnp.int32),
            jax.ShapeDtypeStruct((_NB, 1), jnp.int32),
            jax.ShapeDtypeStruct((1, 1), jnp.int32),
        ),
        out_specs=(
            pl.BlockSpec(memory_space=pltpu.VMEM),
            pl.BlockSpec(memory_space=pltpu.VMEM),
            pl.BlockSpec(memory_space=pltpu.SMEM),
        ),
        interpret=interpret,
    )


def _ffn_body(be_ref, nact_ref, x_ref, w1_ref, b1_ref, w2_ref, b2_ref, o_ref):
    i = pl.program_id(0)

    @pl.when(i < nact_ref[0])
    def _():
        h = jnp.dot(x_ref[...], w1_ref[0], preferred_element_type=jnp.float32)
        h = h + b1_ref[0]
        h = 0.5 * h * (1.0 + lax.erf(h * 0.7071067811865476))
        o = jnp.dot(h, w2_ref[0], preferred_element_type=jnp.float32)
        o_ref[...] = o + b2_ref[0]


def _make_ffn(interpret=False):
    grid_spec = pltpu.PrefetchScalarGridSpec(
        num_scalar_prefetch=2,
        grid=(_NB,),
        in_specs=[
            pl.BlockSpec((_BLK, _DIM), lambda i, be, na: (i, 0)),
            pl.BlockSpec((1, _DIM, _FF), lambda i, be, na: (be[i], 0, 0)),
            pl.BlockSpec((1, 1, _FF), lambda i, be, na: (be[i], 0, 0)),
            pl.BlockSpec((1, _FF, _DIM), lambda i, be, na: (be[i], 0, 0)),
            pl.BlockSpec((1, 1, _DIM), lambda i, be, na: (be[i], 0, 0)),
        ],
        out_specs=pl.BlockSpec((_BLK, _DIM), lambda i, be, na: (i, 0)),
    )
    return pl.pallas_call(
        _ffn_body,
        grid_spec=grid_spec,
        out_shape=jax.ShapeDtypeStruct((_NB * _BLK, _DIM), jnp.float32),
        interpret=interpret,
    )


@functools.cache
def _make_sc_kernels():
    mesh = plsc.VectorSubcoreMesh(core_axis_name="c", subcore_axis_name="s",
                                  num_cores=_NC)
    scratch = [
        pltpu.VMEM((_RPW,), jnp.int32),
        pltpu.VMEM((_RPW, _DIM), jnp.float32),
        pltpu.SemaphoreType.DMA,
    ]

    @functools.partial(
        pl.kernel, mesh=mesh,
        out_type=jax.ShapeDtypeStruct((_NB * _BLK, _DIM), jnp.float32),
        scratch_types=scratch,
    )
    def _sc_scatter_rows(x_hbm, pos_hbm, xs_hbm, idx_v, rows_v, sem):
        wid = lax.axis_index("s") * _NC + lax.axis_index("c")
        base = wid * _RPW
        pltpu.sync_copy(pos_hbm.at[pl.ds(base, _RPW)], idx_v)
        pltpu.sync_copy(x_hbm.at[pl.ds(base, _RPW)], rows_v)
        pltpu.async_copy(rows_v, xs_hbm.at[idx_v], sem).wait()

    @functools.partial(
        pl.kernel, mesh=mesh,
        out_type=jax.ShapeDtypeStruct((_S, _DIM), jnp.float32),
        scratch_types=scratch,
    )
    def _sc_gather_rows(os_hbm, pos_hbm, out_hbm, idx_v, rows_v, sem):
        wid = lax.axis_index("s") * _NC + lax.axis_index("c")
        base = wid * _RPW
        pltpu.sync_copy(pos_hbm.at[pl.ds(base, _RPW)], idx_v)
        pltpu.async_copy(os_hbm.at[idx_v], rows_v, sem).wait()
        pltpu.sync_copy(rows_v, out_hbm.at[pl.ds(base, _RPW)])

    return _sc_scatter_rows, _sc_gather_rows


def kernel(x, route_W, route_b, W1, b1, W2, b2):
    x2 = x.reshape(_S, _DIM)
    wr = jnp.pad(route_W, ((0, 0), (0, _EP - _E)))
    rb = jnp.pad(route_b, (0, _EP - _E))[None, :]
    pos2, be2, nact2 = _make_router()(x2, wr, rb)
    pos = pos2.reshape(_S)
    be = be2.reshape(_NB)
    nact = nact2.reshape(1)
    sc_scatter, sc_gather = _make_sc_kernels()
    xs = sc_scatter(x2, pos)
    os_ = _make_ffn()(be, nact, xs,
                      W1, b1.reshape(_E, 1, _FF),
                      W2, b2.reshape(_E, 1, _DIM))
    out = sc_gather(os_, pos)
    return out.reshape(x.shape)


# speedup vs baseline: 4.6407x; 1.1336x over previous
import functools

import jax
import jax.numpy as jnp
from jax import lax
from jax.experimental import pallas as pl
from jax.experimental.pallas import tpu as pltpu
from jax.experimental.pallas import tpu_sc as plsc

_S, _DIM, _E, _FF = 2048, 1024, 8, 2048
_BLK = 128
_NB = _S // _BLK + _E
_EP = 128
_NC, _NS = 2, 16
_NW = _NC * _NS
_RPW = _S // _NW


def _router_body(x_ref, w_ref, b_ref, pos_ref, boff_ref):
    logits = jnp.dot(x_ref[...].astype(jnp.bfloat16),
                     w_ref[...].astype(jnp.bfloat16),
                     preferred_element_type=jnp.float32) + b_ref[...]
    lane = lax.broadcasted_iota(jnp.int32, (_S, _EP), 1)
    logits = jnp.where(lane < _E, logits, -1e30)
    mx = jnp.max(logits, axis=1, keepdims=True)
    eid = jnp.min(jnp.where(logits == mx, lane, _EP), axis=1, keepdims=True)
    onehot = jnp.where(lane == eid, 1.0, 0.0)

    tri = jnp.where(
        lax.broadcasted_iota(jnp.int32, (_BLK, _BLK), 0)
        >= lax.broadcasted_iota(jnp.int32, (_BLK, _BLK), 1), 1.0, 0.0)

    carry = jnp.zeros((1, _EP), jnp.float32)
    chunks = []
    for g in range(_S // _BLK):
        blk = lax.slice(onehot, (g * _BLK, 0), ((g + 1) * _BLK, _EP))
        c = jnp.dot(tri, blk, preferred_element_type=jnp.float32) + carry
        chunks.append(c)
        carry = lax.slice(c, (_BLK - 1, 0), (_BLK, _EP))
    csum = jnp.concatenate(chunks, axis=0)

    rank = jnp.sum(onehot * csum, axis=1, keepdims=True).astype(jnp.int32) - 1
    counts = lax.slice(csum, (_S - 1, 0), (_S, _EP)).astype(jnp.int32)
    lane_r = lax.broadcasted_iota(jnp.int32, (1, _EP), 1)
    padded = jnp.where(lane_r < _E, ((counts + _BLK - 1) // _BLK) * _BLK, 0)
    triu = jnp.where(
        lax.broadcasted_iota(jnp.int32, (_EP, _EP), 0)
        <= lax.broadcasted_iota(jnp.int32, (_EP, _EP), 1), 1.0, 0.0)
    cum_end = jnp.dot(padded.astype(jnp.float32), triu,
                      preferred_element_type=jnp.float32).astype(jnp.int32)
    off = (cum_end - padded).astype(jnp.float32)
    pos_ref[...] = (jnp.sum(onehot * off, axis=1, keepdims=True).astype(jnp.int32)
                    + rank)

    rowi9 = lax.broadcasted_iota(jnp.int32, (_E + 1, _EP), 0)
    lane9 = lax.broadcasted_iota(jnp.int32, (_E + 1, _EP), 1)
    boff = jnp.sum(jnp.where(lane9 < rowi9,
                             jnp.broadcast_to(padded, (_E + 1, _EP)), 0),
                   axis=1, keepdims=True) // _BLK
    boff_ref[...] = boff.astype(jnp.int32)


def _make_router(interpret=False):
    return pl.pallas_call(
        _router_body,
        out_shape=(
            jax.ShapeDtypeStruct((_S, 1), jnp.int32),
            jax.ShapeDtypeStruct((_E + 1, 1), jnp.int32),
        ),
        out_specs=(
            pl.BlockSpec(memory_space=pltpu.VMEM),
            pl.BlockSpec(memory_space=pltpu.VMEM),
        ),
        interpret=interpret,
    )


def _ffn_body(boff_ref, xs_ref, w1_ref, b1_ref, w2_ref, b2_ref, o_ref):
    e = pl.program_id(0)
    t = pl.program_id(1)
    start = boff_ref[e]
    nblk = boff_ref[e + 1] - start

    def body(j, carry):
        base = pl.multiple_of((start + j) * _BLK, _BLK)
        xb = xs_ref[pl.ds(base, _BLK), :]
        h = jnp.dot(xb, w1_ref[0], preferred_element_type=jnp.float32)
        h = h + b1_ref[0]
        h = 0.5 * h * (1.0 + lax.erf(h * 0.7071067811865476))
        part = jnp.dot(h, w2_ref[0], preferred_element_type=jnp.float32)

        @pl.when(t == 0)
        def _():
            o_ref[pl.ds(base, _BLK), :] = part

        @pl.when(t == 1)
        def _():
            o_ref[pl.ds(base, _BLK), :] = (
                o_ref[pl.ds(base, _BLK), :] + part + b2_ref[0])

        return carry

    lax.fori_loop(0, nblk, body, 0)


def _make_ffn(interpret=False):
    grid_spec = pltpu.PrefetchScalarGridSpec(
        num_scalar_prefetch=1,
        grid=(_E, 2),
        in_specs=[
            pl.BlockSpec((_NB * _BLK, _DIM), lambda e, t, boff: (0, 0)),
            pl.BlockSpec((1, _DIM, _FF // 2), lambda e, t, boff: (e, 0, t)),
            pl.BlockSpec((1, 1, _FF // 2), lambda e, t, boff: (e, 0, t)),
            pl.BlockSpec((1, _FF // 2, _DIM), lambda e, t, boff: (e, t, 0)),
            pl.BlockSpec((1, 1, _DIM), lambda e, t, boff: (e, 0, 0)),
        ],
        out_specs=pl.BlockSpec((_NB * _BLK, _DIM), lambda e, t, boff: (0, 0)),
    )
    return pl.pallas_call(
        _ffn_body,
        grid_spec=grid_spec,
        out_shape=jax.ShapeDtypeStruct((_NB * _BLK, _DIM), jnp.float32),
        interpret=interpret,
    )


@functools.cache
def _make_sc_kernels():
    mesh = plsc.VectorSubcoreMesh(core_axis_name="c", subcore_axis_name="s",
                                  num_cores=_NC)
    scratch = [
        pltpu.VMEM((_RPW,), jnp.int32),
        pltpu.VMEM((_RPW, _DIM), jnp.float32),
        pltpu.SemaphoreType.DMA,
    ]

    @functools.partial(
        pl.kernel, mesh=mesh,
        out_type=jax.ShapeDtypeStruct((_NB * _BLK, _DIM), jnp.float32),
        scratch_types=scratch,
    )
    def _sc_scatter_rows(x_hbm, pos_hbm, xs_hbm, idx_v, rows_v, sem):
        wid = lax.axis_index("s") * _NC + lax.axis_index("c")
        base = wid * _RPW
        pltpu.sync_copy(pos_hbm.at[pl.ds(base, _RPW)], idx_v)
        pltpu.sync_copy(x_hbm.at[pl.ds(base, _RPW)], rows_v)
        pltpu.async_copy(rows_v, xs_hbm.at[idx_v], sem).wait()

    @functools.partial(
        pl.kernel, mesh=mesh,
        out_type=jax.ShapeDtypeStruct((_S, _DIM), jnp.float32),
        scratch_types=scratch,
    )
    def _sc_gather_rows(os_hbm, pos_hbm, out_hbm, idx_v, rows_v, sem):
        wid = lax.axis_index("s") * _NC + lax.axis_index("c")
        base = wid * _RPW
        pltpu.sync_copy(pos_hbm.at[pl.ds(base, _RPW)], idx_v)
        pltpu.async_copy(os_hbm.at[idx_v], rows_v, sem).wait()
        pltpu.sync_copy(rows_v, out_hbm.at[pl.ds(base, _RPW)])

    return _sc_scatter_rows, _sc_gather_rows


def kernel(x, route_W, route_b, W1, b1, W2, b2):
    x2 = x.reshape(_S, _DIM)
    wr = jnp.pad(route_W, ((0, 0), (0, _EP - _E)))
    rb = jnp.pad(route_b, (0, _EP - _E))[None, :]
    pos2, boff2 = _make_router()(x2, wr, rb)
    pos = pos2.reshape(_S)
    boff = boff2.reshape(_E + 1)
    sc_scatter, sc_gather = _make_sc_kernels()
    xs = sc_scatter(x2, pos)
    os_ = _make_ffn()(boff, xs,
                      W1, b1.reshape(_E, 1, _FF),
                      W2, b2.reshape(_E, 1, _DIM))
    out = sc_gather(os_, pos)
    return out.reshape(x.shape)
